# baseline (speedup 1.0000x reference)
import functools

import jax
import jax.numpy as jnp
from jax import lax
from jax.experimental import pallas as pl
from jax.experimental.pallas import tpu as pltpu
from jax.experimental.pallas import tpu_sc as plsc

_T = 16384
_E = 64
_K = 8
_NC = 2
_NS = 16
_NW = _NC * _NS
_TPW = _T // _NW


def _body(logits_hbm, out_w_hbm, out_i_hbm, vals_v, w_v, i_v):
  wid = lax.axis_index("s") * _NC + lax.axis_index("c")
  base = wid * _TPW
  pltpu.sync_copy(logits_hbm.at[pl.ds(base, _TPW)], vals_v)

  lanes = lax.iota(jnp.int32, 16)
  m8 = lanes < _K

  def comb(ak, av, bk, bv):
    rbk = lax.rev(bk, (0,))
    rbv = lax.rev(bv, (0,))
    take_a = ak >= rbk
    return jnp.maximum(ak, rbk), jnp.where(take_a, av, rbv)

  def tok(t, carry):
    ks = []
    vs = []
    for j in range(4):
      kj = vals_v[t, pl.ds(j * 16, 16)]
      sk, sv = plsc.sort_key_val(kj, lanes + j * 16, descending=True)
      ks.append(sk)
      vs.append(sv)
    m01k, m01v = comb(ks[0], vs[0], ks[1], vs[1])
    m23k, m23v = comb(ks[2], vs[2], ks[3], vs[3])
    t01k, t01v = plsc.sort_key_val(m01k, m01v, descending=True)
    t23k, t23v = plsc.sort_key_val(m23k, m23v, descending=True)
    fk0, fv0 = comb(t01k, t01v, t23k, t23v)
    fk, fv = plsc.sort_key_val(fk0, fv0, descending=True)
    ssum = jnp.sum(jnp.where(m8, fk, 0.0))
    w = fk / ssum
    off = pl.multiple_of(t * _K, 8)
    plsc.store_compressed(w_v.at[pl.ds(off, 16)], w, mask=m8)
    plsc.store_compressed(i_v.at[pl.ds(off, 16)], fv, mask=m8)
    return carry

  lax.fori_loop(0, _TPW, tok, 0)

  n = _TPW * _K
  pltpu.sync_copy(w_v.at[pl.ds(0, n)], out_w_hbm.at[pl.ds(base * _K, n)])
  pltpu.sync_copy(i_v.at[pl.ds(0, n)], out_i_hbm.at[pl.ds(base * _K, n)])


_mesh = plsc.VectorSubcoreMesh(
    core_axis_name="c", subcore_axis_name="s", num_cores=_NC, num_subcores=_NS)

_topk_call = pl.kernel(
    _body,
    out_type=(
        jax.ShapeDtypeStruct((_T * _K,), jnp.float32),
        jax.ShapeDtypeStruct((_T * _K,), jnp.int32),
    ),
    mesh=_mesh,
    scratch_types=[
        pltpu.VMEM((_TPW, _E), jnp.float32),
        pltpu.VMEM((_TPW * _K + 8,), jnp.float32),
        pltpu.VMEM((_TPW * _K + 8,), jnp.int32),
    ],
    compiler_params=pltpu.CompilerParams(needs_layout_passes=False),
)


@jax.jit
def kernel(router_logits):
  w, i = _topk_call(router_logits.astype(jnp.float32))
  return w.reshape(_T, _K), i.reshape(_T, _K)


# speedup vs baseline: 1.3590x; 1.3590x over previous
import functools

import jax
import jax.numpy as jnp
from jax import lax
from jax.experimental import pallas as pl
from jax.experimental.pallas import tpu as pltpu
from jax.experimental.pallas import tpu_sc as plsc

_T = 16384
_E = 64
_K = 8
_NC = 2
_NS = 16
_NW = _NC * _NS
_TPW = _T // _NW


def _body(logits_hbm, out_w_hbm, out_i_hbm, vals_v, w_v, i_v):
  wid = lax.axis_index("s") * _NC + lax.axis_index("c")
  base = wid * _TPW
  pltpu.sync_copy(logits_hbm.at[pl.ds(base, _TPW)], vals_v)

  lanes = lax.iota(jnp.int32, 16)
  m8 = lanes < _K

  def comb(ak, av, bk, bv):
    rbk = lax.rev(bk, (0,))
    rbv = lax.rev(bv, (0,))
    take_a = ak >= rbk
    return jnp.maximum(ak, rbk), jnp.where(take_a, av, rbv)

  @plsc.parallel_loop(0, _TPW, unroll=4)
  def tok(t):
    ks = []
    vs = []
    for j in range(4):
      kj = vals_v[t, pl.ds(j * 16, 16)]
      sk, sv = plsc.sort_key_val(kj, lanes + j * 16, descending=True)
      ks.append(sk)
      vs.append(sv)
    m01k, m01v = comb(ks[0], vs[0], ks[1], vs[1])
    m23k, m23v = comb(ks[2], vs[2], ks[3], vs[3])
    t01k, t01v = plsc.sort_key_val(m01k, m01v, descending=True)
    t23k, t23v = plsc.sort_key_val(m23k, m23v, descending=True)
    fk0, fv0 = comb(t01k, t01v, t23k, t23v)
    fk, fv = plsc.sort_key_val(fk0, fv0, descending=True)
    ssum = jnp.sum(jnp.where(m8, fk, 0.0))
    w = fk / ssum
    off = pl.multiple_of(t * _K, 8)
    plsc.store_compressed(w_v.at[pl.ds(off, 16)], w, mask=m8)
    plsc.store_compressed(i_v.at[pl.ds(off, 16)], fv, mask=m8)

  n = _TPW * _K
  pltpu.sync_copy(w_v.at[pl.ds(0, n)], out_w_hbm.at[pl.ds(base * _K, n)])
  pltpu.sync_copy(i_v.at[pl.ds(0, n)], out_i_hbm.at[pl.ds(base * _K, n)])


_mesh = plsc.VectorSubcoreMesh(
    core_axis_name="c", subcore_axis_name="s", num_cores=_NC, num_subcores=_NS)

_topk_call = pl.kernel(
    _body,
    out_type=(
        jax.ShapeDtypeStruct((_T * _K,), jnp.float32),
        jax.ShapeDtypeStruct((_T * _K,), jnp.int32),
    ),
    mesh=_mesh,
    scratch_types=[
        pltpu.VMEM((_TPW, _E), jnp.float32),
        pltpu.VMEM((_TPW * _K + 8,), jnp.float32),
        pltpu.VMEM((_TPW * _K + 8,), jnp.int32),
    ],
    compiler_params=pltpu.CompilerParams(needs_layout_passes=False),
)


@jax.jit
def kernel(router_logits):
  w, i = _topk_call(router_logits.astype(jnp.float32))
  return w.reshape(_T, _K), i.reshape(_T, _K)
